# baseline (speedup 1.0000x reference)
import functools

import jax
import jax.numpy as jnp
from jax import lax
from jax.experimental import pallas as pl
from jax.experimental.pallas import tpu as pltpu
from jax.experimental.pallas import tpu_sc as plsc

N = 10000
E = 320000
D = 128

NC = 2
NS = 16
NW = NC * NS
EPW = E // NW
K = 128
CH = EPW // K
KT = EPW - CH * K
N_PAD = 10240
RPT = N_PAD // NS


def _sc_body(x_hbm, src_hbm, dst_hbm, w_hbm, z_hbm,
             src_all, dst0, dst1, w0, w1, rows0, rows1, dstT, wT, rowsT, acc,
             gsem0, gsem1, dsem0, dsem1):
    c = lax.axis_index("c")
    s = lax.axis_index("s")
    wid = s * NC + c
    ebase = pl.multiple_of(wid * EPW, 8)

    pltpu.sync_copy(src_hbm.at[pl.ds(ebase, EPW)], src_all)

    zero = jnp.zeros((16,), jnp.float32)

    def zrow(i, _):
        for j in range(D // 16):
            rows0[i, pl.ds(j * 16, 16)] = zero
        return 0

    lax.fori_loop(0, K, zrow, 0)
    zbase = s * RPT
    for t in range(RPT // K):
        pltpu.sync_copy(rows0, acc.at[pl.ds(zbase + t * K, K)])
    plsc.subcore_barrier()

    def issue(ci, dst_v, w_v, rows, gsem, dsem):
        off = pl.multiple_of(ebase + ci * K, 8)
        pltpu.async_copy(dst_hbm.at[pl.ds(off, K)], dst_v, dsem)
        pltpu.async_copy(w_hbm.at[pl.ds(off, K)], w_v, dsem)
        pltpu.async_copy(x_hbm.at[src_all.at[pl.ds(ci * K, K)]], rows, gsem)

    def wait(ci, dst_v, w_v, rows, gsem, dsem):
        off = pl.multiple_of(ebase + ci * K, 8)
        pltpu.make_async_copy(dst_hbm.at[pl.ds(off, K)], dst_v, dsem).wait()
        pltpu.make_async_copy(w_hbm.at[pl.ds(off, K)], w_v, dsem).wait()
        pltpu.make_async_copy(
            x_hbm.at[src_all.at[pl.ds(ci * K, K)]], rows, gsem).wait()

    def scale_scatter(dst_v, w_v, rows):
        def sgroup(g, _):
            wvec = w_v[pl.ds(g * 16, 16)]
            for e in range(16):
                wk = wvec[e]
                row = g * 16 + e
                for j in range(D // 16):
                    sl = pl.ds(j * 16, 16)
                    rows[row, sl] = rows[row, sl] * wk
            return 0

        lax.fori_loop(0, K // 16, sgroup, 0)
        pltpu.sync_copy(rows, acc.at[dst_v], add=True)

    issue(0, dst0, w0, rows0, gsem0, dsem0)
    issue(1, dst1, w1, rows1, gsem1, dsem1)

    def pair(p, _):
        c0 = p * 2
        c1 = c0 + 1
        wait(c0, dst0, w0, rows0, gsem0, dsem0)
        scale_scatter(dst0, w0, rows0)

        @pl.when(c0 + 2 <= CH - 1)
        def _():
            issue(c0 + 2, dst0, w0, rows0, gsem0, dsem0)

        wait(c1, dst1, w1, rows1, gsem1, dsem1)
        scale_scatter(dst1, w1, rows1)

        @pl.when(c1 + 2 <= CH - 1)
        def _():
            issue(c1 + 2, dst1, w1, rows1, gsem1, dsem1)

        return 0

    lax.fori_loop(0, CH // 2, pair, 0)

    offT = pl.multiple_of(ebase + CH * K, 8)
    pltpu.async_copy(dst_hbm.at[pl.ds(offT, KT)], dstT, dsem0)
    pltpu.async_copy(w_hbm.at[pl.ds(offT, KT)], wT, dsem0)
    pltpu.async_copy(x_hbm.at[src_all.at[pl.ds(CH * K, KT)]], rowsT, gsem0)
    pltpu.make_async_copy(dst_hbm.at[pl.ds(offT, KT)], dstT, dsem0).wait()
    pltpu.make_async_copy(w_hbm.at[pl.ds(offT, KT)], wT, dsem0).wait()
    pltpu.make_async_copy(
        x_hbm.at[src_all.at[pl.ds(CH * K, KT)]], rowsT, gsem0).wait()
    wvecT = wT[pl.ds(0, 16)]
    for e in range(KT):
        wk = wvecT[e]
        for j in range(D // 16):
            sl = pl.ds(j * 16, 16)
            rowsT[e, sl] = rowsT[e, sl] * wk
    pltpu.sync_copy(rowsT, acc.at[dstT], add=True)
    plsc.subcore_barrier()

    pltpu.sync_copy(acc.at[pl.ds(zbase, RPT)], z_hbm.at[c, pl.ds(zbase, RPT)])


_sc_aggregate = functools.partial(
    pl.kernel,
    out_type=jax.ShapeDtypeStruct((NC, N_PAD, D), jnp.float32),
    mesh=plsc.VectorSubcoreMesh(core_axis_name="c", subcore_axis_name="s"),
    scratch_types=[
        pltpu.VMEM((EPW,), jnp.int32),
        pltpu.VMEM((K,), jnp.int32),
        pltpu.VMEM((K,), jnp.int32),
        pltpu.VMEM((K,), jnp.float32),
        pltpu.VMEM((K,), jnp.float32),
        pltpu.VMEM((K, D), jnp.float32),
        pltpu.VMEM((K, D), jnp.float32),
        pltpu.VMEM((KT,), jnp.int32),
        pltpu.VMEM((KT,), jnp.float32),
        pltpu.VMEM((KT, D), jnp.float32),
        pltpu.VMEM_SHARED((N_PAD, D), jnp.float32),
        pltpu.SemaphoreType.DMA,
        pltpu.SemaphoreType.DMA,
        pltpu.SemaphoreType.DMA,
        pltpu.SemaphoreType.DMA,
    ],
)(_sc_body)


def _tc_body(z_ref, w_ref, b_ref, o_ref):
    z = z_ref[0] + z_ref[1]
    o_ref[...] = (jnp.dot(z, w_ref[...], preferred_element_type=jnp.float32)
                  + b_ref[...])


_TC_BLK = 1000


def _tc_matmul(z, W, b2):
    return pl.pallas_call(
        _tc_body,
        grid=(N // _TC_BLK,),
        in_specs=[
            pl.BlockSpec((NC, _TC_BLK, D), lambda i: (0, i, 0)),
            pl.BlockSpec((D, D), lambda i: (0, 0)),
            pl.BlockSpec((1, D), lambda i: (0, 0)),
        ],
        out_specs=pl.BlockSpec((_TC_BLK, D), lambda i: (i, 0)),
        out_shape=jax.ShapeDtypeStruct((N, D), jnp.float32),
    )(z, W, b2)


def kernel(x, edge_index, edge_weight, W, b):
    src = edge_index[0].astype(jnp.int32)
    dst = edge_index[1].astype(jnp.int32)
    z = _sc_aggregate(x, src, dst, edge_weight)
    return _tc_matmul(z, W, b.reshape(1, D))


# speedup vs baseline: 1.0121x; 1.0121x over previous
import functools

import jax
import jax.numpy as jnp
from jax import lax
from jax.experimental import pallas as pl
from jax.experimental.pallas import tpu as pltpu
from jax.experimental.pallas import tpu_sc as plsc

N = 10000
E = 320000
D = 128

NC = 2
NS = 16
NW = NC * NS
EPW = E // NW
K = 128
KH = K // 2
CH = EPW // K
KT = EPW - CH * K
N_PAD = 10240
RPT = N_PAD // NS


def _sc_body(x_hbm, src_hbm, dst_hbm, w_hbm, z_hbm,
             src_all, dst0, dst1, w0, w1, rows0, rows1, dstT, wT, rowsT, acc,
             gsem0, gsem1, dsem0, dsem1, ssem0, ssem1):
    c = lax.axis_index("c")
    s = lax.axis_index("s")
    wid = s * NC + c
    ebase = pl.multiple_of(wid * EPW, 8)

    pltpu.sync_copy(src_hbm.at[pl.ds(ebase, EPW)], src_all)

    zero = jnp.zeros((16,), jnp.float32)

    def zrow(i, _):
        for j in range(D // 16):
            rows0[i, pl.ds(j * 16, 16)] = zero
        return 0

    lax.fori_loop(0, K, zrow, 0)
    zbase = s * RPT
    for t in range(RPT // K):
        pltpu.sync_copy(rows0, acc.at[pl.ds(zbase + t * K, K)])
    plsc.subcore_barrier()

    def issue(ci, dst_v, w_v, rows, gsem, dsem):
        off = pl.multiple_of(ebase + ci * K, 8)
        pltpu.async_copy(dst_hbm.at[pl.ds(off, KH)], dst_v.at[0], dsem)
        pltpu.async_copy(dst_hbm.at[pl.ds(off + KH, KH)], dst_v.at[1], dsem)
        pltpu.async_copy(w_hbm.at[pl.ds(off, K)], w_v, dsem)
        pltpu.async_copy(x_hbm.at[src_all.at[pl.ds(ci * K, K)]], rows, gsem)

    def wait(ci, dst_v, w_v, rows, gsem, dsem):
        off = pl.multiple_of(ebase + ci * K, 8)
        pltpu.make_async_copy(
            dst_hbm.at[pl.ds(off, KH)], dst_v.at[0], dsem).wait()
        pltpu.make_async_copy(
            dst_hbm.at[pl.ds(off + KH, KH)], dst_v.at[1], dsem).wait()
        pltpu.make_async_copy(w_hbm.at[pl.ds(off, K)], w_v, dsem).wait()
        pltpu.make_async_copy(
            x_hbm.at[src_all.at[pl.ds(ci * K, K)]], rows, gsem).wait()

    def scale_scatter(dst_v, w_v, rows, ssem):
        def sgroup(g, _):
            wvec = w_v[pl.ds(g * 16, 16)]
            for e in range(16):
                wk = wvec[e]
                row = g * 16 + e
                for j in range(D // 16):
                    sl = pl.ds(j * 16, 16)
                    rows[row, sl] = rows[row, sl] * wk
            return 0

        lax.fori_loop(0, KH // 16, sgroup, 0)
        pltpu.async_copy(rows.at[pl.ds(0, KH)], acc.at[dst_v.at[0]], ssem,
                         add=True)
        lax.fori_loop(KH // 16, K // 16, sgroup, 0)
        pltpu.sync_copy(rows.at[pl.ds(KH, KH)], acc.at[dst_v.at[1]],
                        add=True)
        pltpu.make_async_copy(rows.at[pl.ds(0, KH)], acc.at[dst_v.at[0]],
                              ssem).wait()

    issue(0, dst0, w0, rows0, gsem0, dsem0)
    issue(1, dst1, w1, rows1, gsem1, dsem1)

    def pair(p, _):
        c0 = p * 2
        c1 = c0 + 1
        wait(c0, dst0, w0, rows0, gsem0, dsem0)
        scale_scatter(dst0, w0, rows0, ssem0)

        @pl.when(c0 + 2 <= CH - 1)
        def _():
            issue(c0 + 2, dst0, w0, rows0, gsem0, dsem0)

        wait(c1, dst1, w1, rows1, gsem1, dsem1)
        scale_scatter(dst1, w1, rows1, ssem1)

        @pl.when(c1 + 2 <= CH - 1)
        def _():
            issue(c1 + 2, dst1, w1, rows1, gsem1, dsem1)

        return 0

    lax.fori_loop(0, CH // 2, pair, 0)

    offT = pl.multiple_of(ebase + CH * K, 8)
    pltpu.async_copy(dst_hbm.at[pl.ds(offT, KT)], dstT, dsem0)
    pltpu.async_copy(w_hbm.at[pl.ds(offT, KT)], wT, dsem0)
    pltpu.async_copy(x_hbm.at[src_all.at[pl.ds(CH * K, KT)]], rowsT, gsem0)
    pltpu.make_async_copy(dst_hbm.at[pl.ds(offT, KT)], dstT, dsem0).wait()
    pltpu.make_async_copy(w_hbm.at[pl.ds(offT, KT)], wT, dsem0).wait()
    pltpu.make_async_copy(
        x_hbm.at[src_all.at[pl.ds(CH * K, KT)]], rowsT, gsem0).wait()
    wvecT = wT[pl.ds(0, 16)]
    for e in range(KT):
        wk = wvecT[e]
        for j in range(D // 16):
            sl = pl.ds(j * 16, 16)
            rowsT[e, sl] = rowsT[e, sl] * wk
    pltpu.sync_copy(rowsT, acc.at[dstT], add=True)
    plsc.subcore_barrier()

    pltpu.sync_copy(acc.at[pl.ds(zbase, RPT)], z_hbm.at[c, pl.ds(zbase, RPT)])


_sc_aggregate = functools.partial(
    pl.kernel,
    out_type=jax.ShapeDtypeStruct((NC, N_PAD, D), jnp.float32),
    mesh=plsc.VectorSubcoreMesh(core_axis_name="c", subcore_axis_name="s"),
    scratch_types=[
        pltpu.VMEM((EPW,), jnp.int32),
        pltpu.VMEM((2, KH), jnp.int32),
        pltpu.VMEM((2, KH), jnp.int32),
        pltpu.VMEM((K,), jnp.float32),
        pltpu.VMEM((K,), jnp.float32),
        pltpu.VMEM((K, D), jnp.float32),
        pltpu.VMEM((K, D), jnp.float32),
        pltpu.VMEM((KT,), jnp.int32),
        pltpu.VMEM((KT,), jnp.float32),
        pltpu.VMEM((KT, D), jnp.float32),
        pltpu.VMEM_SHARED((N_PAD, D), jnp.float32),
        pltpu.SemaphoreType.DMA,
        pltpu.SemaphoreType.DMA,
        pltpu.SemaphoreType.DMA,
        pltpu.SemaphoreType.DMA,
        pltpu.SemaphoreType.DMA,
        pltpu.SemaphoreType.DMA,
    ],
)(_sc_body)


def _tc_body(z_ref, w_ref, b_ref, o_ref):
    z = z_ref[0] + z_ref[1]
    o_ref[...] = (jnp.dot(z, w_ref[...], preferred_element_type=jnp.float32)
                  + b_ref[...])


_TC_BLK = 1000


def _tc_matmul(z, W, b2):
    return pl.pallas_call(
        _tc_body,
        grid=(N // _TC_BLK,),
        in_specs=[
            pl.BlockSpec((NC, _TC_BLK, D), lambda i: (0, i, 0)),
            pl.BlockSpec((D, D), lambda i: (0, 0)),
            pl.BlockSpec((1, D), lambda i: (0, 0)),
        ],
        out_specs=pl.BlockSpec((_TC_BLK, D), lambda i: (i, 0)),
        out_shape=jax.ShapeDtypeStruct((N, D), jnp.float32),
    )(z, W, b2)


def kernel(x, edge_index, edge_weight, W, b):
    src = edge_index[0].astype(jnp.int32)
    dst = edge_index[1].astype(jnp.int32)
    z = _sc_aggregate(x, src, dst, edge_weight)
    return _tc_matmul(z, W, b.reshape(1, D))


# speedup vs baseline: 1.0167x; 1.0046x over previous
import functools

import jax
import jax.numpy as jnp
from jax import lax
from jax.experimental import pallas as pl
from jax.experimental.pallas import tpu as pltpu
from jax.experimental.pallas import tpu_sc as plsc

N = 10000
E = 320000
D = 128

NC = 2
NS = 16
NW = NC * NS
EPW = E // NW
K = 128
KH = K // 2
CH = EPW // K
KT = EPW - CH * K
N_PAD = 10240
RPT = N_PAD // NS


def _sc_body(x_hbm, src_hbm, dst_hbm, w_hbm, z_hbm,
             src_all, dst0, dst1, w0, w1, rows0, rows1, dstT, wT, rowsT, acc,
             gsem0, gsem1, dsem0, dsem1, ssem0, ssem1):
    c = lax.axis_index("c")
    s = lax.axis_index("s")
    wid = s * NC + c
    ebase = pl.multiple_of(wid * EPW, 8)

    pltpu.sync_copy(src_hbm.at[pl.ds(ebase, EPW)], src_all)

    zero = jnp.zeros((16,), jnp.float32)

    def zrow(i, _):
        for j in range(D // 16):
            rows0[i, pl.ds(j * 16, 16)] = zero
        return 0

    lax.fori_loop(0, K, zrow, 0)
    zbase = s * RPT
    for t in range(RPT // K):
        pltpu.async_copy(rows0, acc.at[pl.ds(zbase + t * K, K)], ssem0)
    for t in range(RPT // K):
        pltpu.make_async_copy(
            rows0, acc.at[pl.ds(zbase + t * K, K)], ssem0).wait()
    plsc.subcore_barrier()

    def issue(ci, dst_v, w_v, rows, gsem, dsem):
        off = pl.multiple_of(ebase + ci * K, 8)
        pltpu.async_copy(dst_hbm.at[pl.ds(off, KH)], dst_v.at[0], dsem)
        pltpu.async_copy(dst_hbm.at[pl.ds(off + KH, KH)], dst_v.at[1], dsem)
        pltpu.async_copy(w_hbm.at[pl.ds(off, K)], w_v, dsem)
        pltpu.async_copy(x_hbm.at[src_all.at[pl.ds(ci * K, K)]], rows, gsem)

    def wait(ci, dst_v, w_v, rows, gsem, dsem):
        off = pl.multiple_of(ebase + ci * K, 8)
        pltpu.make_async_copy(
            dst_hbm.at[pl.ds(off, KH)], dst_v.at[0], dsem).wait()
        pltpu.make_async_copy(
            dst_hbm.at[pl.ds(off + KH, KH)], dst_v.at[1], dsem).wait()
        pltpu.make_async_copy(w_hbm.at[pl.ds(off, K)], w_v, dsem).wait()
        pltpu.make_async_copy(
            x_hbm.at[src_all.at[pl.ds(ci * K, K)]], rows, gsem).wait()

    def scale_scatter(dst_v, w_v, rows, ssem):
        def sgroup(g, _):
            wvec = w_v[pl.ds(g * 16, 16)]
            for e in range(16):
                wk = wvec[e]
                row = g * 16 + e
                for j in range(D // 16):
                    sl = pl.ds(j * 16, 16)
                    rows[row, sl] = rows[row, sl] * wk
            return 0

        lax.fori_loop(0, KH // 16, sgroup, 0)
        pltpu.async_copy(rows.at[pl.ds(0, KH)], acc.at[dst_v.at[0]], ssem,
                         add=True)
        lax.fori_loop(KH // 16, K // 16, sgroup, 0)
        pltpu.sync_copy(rows.at[pl.ds(KH, KH)], acc.at[dst_v.at[1]],
                        add=True)
        pltpu.make_async_copy(rows.at[pl.ds(0, KH)], acc.at[dst_v.at[0]],
                              ssem).wait()

    issue(0, dst0, w0, rows0, gsem0, dsem0)
    issue(1, dst1, w1, rows1, gsem1, dsem1)

    def pair(p, _):
        c0 = p * 2
        c1 = c0 + 1
        wait(c0, dst0, w0, rows0, gsem0, dsem0)
        scale_scatter(dst0, w0, rows0, ssem0)

        @pl.when(c0 + 2 <= CH - 1)
        def _():
            issue(c0 + 2, dst0, w0, rows0, gsem0, dsem0)

        wait(c1, dst1, w1, rows1, gsem1, dsem1)
        scale_scatter(dst1, w1, rows1, ssem1)

        @pl.when(c1 + 2 <= CH - 1)
        def _():
            issue(c1 + 2, dst1, w1, rows1, gsem1, dsem1)

        return 0

    lax.fori_loop(0, CH // 2, pair, 0)

    offT = pl.multiple_of(ebase + CH * K, 8)
    pltpu.async_copy(dst_hbm.at[pl.ds(offT, KT)], dstT, dsem0)
    pltpu.async_copy(w_hbm.at[pl.ds(offT, KT)], wT, dsem0)
    pltpu.async_copy(x_hbm.at[src_all.at[pl.ds(CH * K, KT)]], rowsT, gsem0)
    pltpu.make_async_copy(dst_hbm.at[pl.ds(offT, KT)], dstT, dsem0).wait()
    pltpu.make_async_copy(w_hbm.at[pl.ds(offT, KT)], wT, dsem0).wait()
    pltpu.make_async_copy(
        x_hbm.at[src_all.at[pl.ds(CH * K, KT)]], rowsT, gsem0).wait()
    wvecT = wT[pl.ds(0, 16)]
    for e in range(KT):
        wk = wvecT[e]
        for j in range(D // 16):
            sl = pl.ds(j * 16, 16)
            rowsT[e, sl] = rowsT[e, sl] * wk
    pltpu.sync_copy(rowsT, acc.at[dstT], add=True)
    plsc.subcore_barrier()

    pltpu.sync_copy(acc.at[pl.ds(zbase, RPT)], z_hbm.at[c, pl.ds(zbase, RPT)])


_sc_aggregate = functools.partial(
    pl.kernel,
    out_type=jax.ShapeDtypeStruct((NC, N_PAD, D), jnp.float32),
    mesh=plsc.VectorSubcoreMesh(core_axis_name="c", subcore_axis_name="s"),
    scratch_types=[
        pltpu.VMEM((EPW,), jnp.int32),
        pltpu.VMEM((2, KH), jnp.int32),
        pltpu.VMEM((2, KH), jnp.int32),
        pltpu.VMEM((K,), jnp.float32),
        pltpu.VMEM((K,), jnp.float32),
        pltpu.VMEM((K, D), jnp.float32),
        pltpu.VMEM((K, D), jnp.float32),
        pltpu.VMEM((KT,), jnp.int32),
        pltpu.VMEM((KT,), jnp.float32),
        pltpu.VMEM((KT, D), jnp.float32),
        pltpu.VMEM_SHARED((N_PAD, D), jnp.float32),
        pltpu.SemaphoreType.DMA,
        pltpu.SemaphoreType.DMA,
        pltpu.SemaphoreType.DMA,
        pltpu.SemaphoreType.DMA,
        pltpu.SemaphoreType.DMA,
        pltpu.SemaphoreType.DMA,
    ],
)(_sc_body)


def _tc_body(z_ref, w_ref, b_ref, o_ref):
    z = z_ref[0] + z_ref[1]
    o_ref[...] = (jnp.dot(z, w_ref[...], preferred_element_type=jnp.float32)
                  + b_ref[...])


_TC_BLK = 1000


def _tc_matmul(z, W, b2):
    return pl.pallas_call(
        _tc_body,
        grid=(N // _TC_BLK,),
        in_specs=[
            pl.BlockSpec((NC, _TC_BLK, D), lambda i: (0, i, 0)),
            pl.BlockSpec((D, D), lambda i: (0, 0)),
            pl.BlockSpec((1, D), lambda i: (0, 0)),
        ],
        out_specs=pl.BlockSpec((_TC_BLK, D), lambda i: (i, 0)),
        out_shape=jax.ShapeDtypeStruct((N, D), jnp.float32),
    )(z, W, b2)


def kernel(x, edge_index, edge_weight, W, b):
    src = edge_index[0].astype(jnp.int32)
    dst = edge_index[1].astype(jnp.int32)
    z = _sc_aggregate(x, src, dst, edge_weight)
    return _tc_matmul(z, W, b.reshape(1, D))


# speedup vs baseline: 1.0841x; 1.0663x over previous
import functools

import jax
import jax.numpy as jnp
from jax import lax
from jax.experimental import pallas as pl
from jax.experimental.pallas import tpu as pltpu
from jax.experimental.pallas import tpu_sc as plsc

N = 10000
E = 320000
D = 128

NC = 2
NS = 16
NW = NC * NS
EPW = E // NW
K = 128
KH = K // 2
CH = EPW // K
KT = EPW - CH * K
N_PAD = 10240
RPT = N_PAD // NS


def _sc_body(x_hbm, ei_hbm, w_hbm, z_hbm,
             src_all, dst0, dst1, w0, w1, rows0, rows1, dstT, wT, rowsT, acc,
             gsem0, gsem1, dsem0, dsem1, ssem0, ssem1):
    c = lax.axis_index("c")
    s = lax.axis_index("s")
    wid = s * NC + c
    ebase = pl.multiple_of(wid * EPW, 8)

    pltpu.sync_copy(ei_hbm.at[pl.ds(ebase, EPW)], src_all)

    zero = jnp.zeros((16,), jnp.float32)

    def zrow(i, _):
        for j in range(D // 16):
            rows0[i, pl.ds(j * 16, 16)] = zero
        return 0

    lax.fori_loop(0, K, zrow, 0)
    zbase = s * RPT
    for t in range(RPT // K):
        pltpu.async_copy(rows0, acc.at[pl.ds(zbase + t * K, K)], ssem0)
    for t in range(RPT // K):
        pltpu.make_async_copy(
            rows0, acc.at[pl.ds(zbase + t * K, K)], ssem0).wait()
    plsc.subcore_barrier()

    def issue(ci, dst_v, w_v, rows, gsem, dsem):
        off = pl.multiple_of(ebase + ci * K, 8)
        pltpu.async_copy(ei_hbm.at[pl.ds(E + off, KH)], dst_v.at[0], dsem)
        pltpu.async_copy(ei_hbm.at[pl.ds(E + off + KH, KH)], dst_v.at[1],
                         dsem)
        pltpu.async_copy(w_hbm.at[pl.ds(off, K)], w_v, dsem)
        pltpu.async_copy(x_hbm.at[src_all.at[pl.ds(ci * K, K)]], rows, gsem)

    def wait(ci, dst_v, w_v, rows, gsem, dsem):
        off = pl.multiple_of(ebase + ci * K, 8)
        pltpu.make_async_copy(
            ei_hbm.at[pl.ds(E + off, KH)], dst_v.at[0], dsem).wait()
        pltpu.make_async_copy(
            ei_hbm.at[pl.ds(E + off + KH, KH)], dst_v.at[1], dsem).wait()
        pltpu.make_async_copy(w_hbm.at[pl.ds(off, K)], w_v, dsem).wait()
        pltpu.make_async_copy(
            x_hbm.at[src_all.at[pl.ds(ci * K, K)]], rows, gsem).wait()

    def scale_scatter(dst_v, w_v, rows, ssem):
        def sgroup(g, _):
            wvec = w_v[pl.ds(g * 16, 16)]
            for e in range(16):
                wk = wvec[e]
                row = g * 16 + e
                for j in range(D // 16):
                    sl = pl.ds(j * 16, 16)
                    rows[row, sl] = rows[row, sl] * wk
            return 0

        lax.fori_loop(0, KH // 16, sgroup, 0)
        pltpu.async_copy(rows.at[pl.ds(0, KH)], acc.at[dst_v.at[0]], ssem,
                         add=True)
        lax.fori_loop(KH // 16, K // 16, sgroup, 0)
        pltpu.sync_copy(rows.at[pl.ds(KH, KH)], acc.at[dst_v.at[1]],
                        add=True)
        pltpu.make_async_copy(rows.at[pl.ds(0, KH)], acc.at[dst_v.at[0]],
                              ssem).wait()

    issue(0, dst0, w0, rows0, gsem0, dsem0)
    issue(1, dst1, w1, rows1, gsem1, dsem1)

    def pair(p, _):
        c0 = p * 2
        c1 = c0 + 1
        wait(c0, dst0, w0, rows0, gsem0, dsem0)
        scale_scatter(dst0, w0, rows0, ssem0)

        @pl.when(c0 + 2 <= CH - 1)
        def _():
            issue(c0 + 2, dst0, w0, rows0, gsem0, dsem0)

        wait(c1, dst1, w1, rows1, gsem1, dsem1)
        scale_scatter(dst1, w1, rows1, ssem1)

        @pl.when(c1 + 2 <= CH - 1)
        def _():
            issue(c1 + 2, dst1, w1, rows1, gsem1, dsem1)

        return 0

    lax.fori_loop(0, CH // 2, pair, 0)

    offT = pl.multiple_of(ebase + CH * K, 8)
    pltpu.async_copy(ei_hbm.at[pl.ds(E + offT, KT)], dstT, dsem0)
    pltpu.async_copy(w_hbm.at[pl.ds(offT, KT)], wT, dsem0)
    pltpu.async_copy(x_hbm.at[src_all.at[pl.ds(CH * K, KT)]], rowsT, gsem0)
    pltpu.make_async_copy(
        ei_hbm.at[pl.ds(E + offT, KT)], dstT, dsem0).wait()
    pltpu.make_async_copy(w_hbm.at[pl.ds(offT, KT)], wT, dsem0).wait()
    pltpu.make_async_copy(
        x_hbm.at[src_all.at[pl.ds(CH * K, KT)]], rowsT, gsem0).wait()
    wvecT = wT[pl.ds(0, 16)]
    for e in range(KT):
        wk = wvecT[e]
        for j in range(D // 16):
            sl = pl.ds(j * 16, 16)
            rowsT[e, sl] = rowsT[e, sl] * wk
    pltpu.sync_copy(rowsT, acc.at[dstT], add=True)
    plsc.subcore_barrier()

    pltpu.sync_copy(acc.at[pl.ds(zbase, RPT)], z_hbm.at[c, pl.ds(zbase, RPT)])


_sc_aggregate = functools.partial(
    pl.kernel,
    out_type=jax.ShapeDtypeStruct((NC, N_PAD, D), jnp.float32),
    mesh=plsc.VectorSubcoreMesh(core_axis_name="c", subcore_axis_name="s"),
    scratch_types=[
        pltpu.VMEM((EPW,), jnp.int32),
        pltpu.VMEM((2, KH), jnp.int32),
        pltpu.VMEM((2, KH), jnp.int32),
        pltpu.VMEM((K,), jnp.float32),
        pltpu.VMEM((K,), jnp.float32),
        pltpu.VMEM((K, D), jnp.float32),
        pltpu.VMEM((K, D), jnp.float32),
        pltpu.VMEM((KT,), jnp.int32),
        pltpu.VMEM((KT,), jnp.float32),
        pltpu.VMEM((KT, D), jnp.float32),
        pltpu.VMEM_SHARED((N_PAD, D), jnp.float32),
        pltpu.SemaphoreType.DMA,
        pltpu.SemaphoreType.DMA,
        pltpu.SemaphoreType.DMA,
        pltpu.SemaphoreType.DMA,
        pltpu.SemaphoreType.DMA,
        pltpu.SemaphoreType.DMA,
    ],
)(_sc_body)


def _tc_body(z_ref, w_ref, b_ref, o_ref):
    z = z_ref[0] + z_ref[1]
    o_ref[...] = (jnp.dot(z, w_ref[...], preferred_element_type=jnp.float32)
                  + b_ref[...])


_TC_BLK = 1000


def _tc_matmul(z, W, b2):
    return pl.pallas_call(
        _tc_body,
        grid=(N // _TC_BLK,),
        in_specs=[
            pl.BlockSpec((NC, _TC_BLK, D), lambda i: (0, i, 0)),
            pl.BlockSpec((D, D), lambda i: (0, 0)),
            pl.BlockSpec((1, D), lambda i: (0, 0)),
        ],
        out_specs=pl.BlockSpec((_TC_BLK, D), lambda i: (i, 0)),
        out_shape=jax.ShapeDtypeStruct((N, D), jnp.float32),
    )(z, W, b2)


def kernel(x, edge_index, edge_weight, W, b):
    ei = edge_index.astype(jnp.int32).reshape(2 * E)
    z = _sc_aggregate(x, ei, edge_weight)
    return _tc_matmul(z, W, b.reshape(1, D))
